# baseline (speedup 1.0000x reference)
import functools
import numpy as np
import jax
import jax.numpy as jnp
from jax import lax
from jax.experimental import pallas as pl
from jax.experimental.pallas import tpu as pltpu
from jax.experimental.pallas import tpu_sc as plsc

H = 4
C = 128
TE = 32
EE = 128
MEM = 128
N = 10000
E = 160000
B = 64
NODE_DIM = EE + MEM
EDGE_DIM = TE + EE
NUM_BLOCK = 3
HC = H * C

NW = 32
EB = 512
NB = 400
NBLK = N // NB
E_PAD = E + EB
PER_W = E_PAD // NW
GCH = 200
G_FULL = PER_W // GCH
G_TAIL = PER_W - G_FULL * GCH

def _sc_gather(table, idx, D):
    @functools.partial(
        pl.kernel,
        mesh=plsc.VectorSubcoreMesh(core_axis_name="c", subcore_axis_name="s"),
        out_type=jax.ShapeDtypeStruct((E_PAD, D), jnp.float32),
        scratch_types=[
            pltpu.VMEM((GCH,), jnp.int32),
            pltpu.VMEM((GCH, D), jnp.float32),
            pltpu.VMEM((G_TAIL,), jnp.int32),
            pltpu.VMEM((G_TAIL, D), jnp.float32),
            pltpu.SemaphoreType.DMA,
        ],
    )
    def k(table_hbm, idx_hbm, out_hbm, idx_v, rows_v, idx_t, rows_t, sem):
        wid = lax.axis_index("s") * 2 + lax.axis_index("c")
        base = wid * PER_W

        def body(j, carry):
            off = base + j * GCH
            pltpu.sync_copy(idx_hbm.at[pl.ds(off, GCH)], idx_v)
            pltpu.async_copy(table_hbm.at[idx_v], rows_v, sem).wait()
            pltpu.sync_copy(rows_v, out_hbm.at[pl.ds(off, GCH)])
            return carry

        lax.fori_loop(0, G_FULL, body, 0)
        toff = base + G_FULL * GCH
        pltpu.sync_copy(idx_hbm.at[pl.ds(toff, G_TAIL)], idx_t)
        pltpu.async_copy(table_hbm.at[idx_t], rows_t, sem).wait()
        pltpu.sync_copy(rows_t, out_hbm.at[pl.ds(toff, G_TAIL)])

    return k(table, idx)


def _node_ts_kernel(b_ref, t_ref, o_ref):
    bid = jax.lax.broadcasted_iota(jnp.int32, (1000, B), 1)
    sel = (b_ref[...] == bid).astype(jnp.float32)
    o_ref[...] = jnp.sum(sel * t_ref[...], axis=1, keepdims=True)


def _node_ts(batch, ts):
    return pl.pallas_call(
        _node_ts_kernel,
        grid=(N // 1000,),
        in_specs=[
            pl.BlockSpec((1000, 1), lambda i: (i, 0)),
            pl.BlockSpec((1, B), lambda i: (0, 0)),
        ],
        out_specs=pl.BlockSpec((1000, 1), lambda i: (i, 0)),
        out_shape=jax.ShapeDtypeStruct((N, 1), jnp.float32),
    )(batch.reshape(N, 1), ts.reshape(1, B))


def _proj_kernel(x_ref, wq_ref, wk_ref, wv_ref, ws_ref, bq_ref, bk_ref,
                 bv_ref, bs_ref, q_ref, k_ref, v_ref, s_ref):
    x = x_ref[...]
    f32 = jnp.float32
    q_ref[...] = jnp.dot(x, wq_ref[...], preferred_element_type=f32) + bq_ref[...]
    k_ref[...] = jnp.dot(x, wk_ref[...], preferred_element_type=f32) + bk_ref[...]
    v_ref[...] = jnp.dot(x, wv_ref[...], preferred_element_type=f32) + bv_ref[...]
    s_ref[...] = jnp.dot(x, ws_ref[...], preferred_element_type=f32) + bs_ref[...]


def _proj(x, wq, wk, wv, ws, bq, bk, bv, bs):
    ind = x.shape[1]
    blk = 1000
    wspec = pl.BlockSpec((ind, HC), lambda i: (0, 0))
    bspec = pl.BlockSpec((1, HC), lambda i: (0, 0))
    ospec = pl.BlockSpec((blk, HC), lambda i: (i, 0))
    oshape = jax.ShapeDtypeStruct((N, HC), jnp.float32)
    return pl.pallas_call(
        _proj_kernel,
        grid=(N // blk,),
        in_specs=[pl.BlockSpec((blk, ind), lambda i: (i, 0)),
                  wspec, wspec, wspec, wspec, bspec, bspec, bspec, bspec],
        out_specs=[ospec, ospec, ospec, ospec],
        out_shape=[oshape, oshape, oshape, oshape],
    )(x, wq, wk, wv, ws, bq.reshape(1, HC), bk.reshape(1, HC),
      bv.reshape(1, HC), bs.reshape(1, HC))


EB2 = 1056


def _eproj_body(ea_ref, ts_ref, tw_ref, tb_ref, wt_ref, wa_ref, be_ref, o_ref):
    ea = ea_ref[...]
    rel_t = ts_ref[...] - ea[:, EE:EE + 1]
    rel_emb = jnp.cos(rel_t * tw_ref[...] + tb_ref[...])
    o_ref[...] = (jnp.dot(rel_emb, wt_ref[...], preferred_element_type=jnp.float32)
                  + jnp.dot(ea[:, :EE], wa_ref[...], preferred_element_type=jnp.float32)
                  + be_ref[...])


def _eproj(ea_ext_s, ts_col, te_w, te_b, We, be):
    return pl.pallas_call(
        _eproj_body,
        grid=(E_PAD // EB2,),
        in_specs=[
            pl.BlockSpec((EB2, 256), lambda i: (i, 0)),
            pl.BlockSpec((EB2, 1), lambda i: (i, 0)),
            pl.BlockSpec((1, TE), lambda i: (0, 0)),
            pl.BlockSpec((1, TE), lambda i: (0, 0)),
            pl.BlockSpec((TE, HC), lambda i: (0, 0)),
            pl.BlockSpec((EE, HC), lambda i: (0, 0)),
            pl.BlockSpec((1, HC), lambda i: (0, 0)),
        ],
        out_specs=pl.BlockSpec((EB2, HC), lambda i: (i, 0)),
        out_shape=jax.ShapeDtypeStruct((E_PAD, HC), jnp.float32),
    )(ea_ext_s, ts_col, te_w.reshape(1, TE), te_b.reshape(1, TE),
      We[:TE], We[TE:], be.reshape(1, HC))


_INV_SQRT_C = np.float32(1.0 / np.sqrt(C))
_NEG = np.float32(-1e30)


def _attn_kernel(offs_ref, q_ref, s_ref, dst_ref, kg_ref, vg_ref, e_ref,
                 o_ref, adump_ref,
                 bd_ref, b1_ref, b2_ref, ba_ref, sem):
    nb = pl.program_id(0)
    off0 = offs_ref[nb]
    off1 = offs_ref[nb + 1]
    astart = (off0 // 8) * 8
    nch = (off1 - astart + EB - 1) // EB
    base = nb * NB

    lane_iota = jax.lax.broadcasted_iota(jnp.int32, (EB, NB), 1)

    def load_chunk(j, big1_src, big2_src):
        start = astart + j * EB
        cp0 = pltpu.make_async_copy(dst_ref.at[pl.ds(start, EB)], bd_ref, sem)
        cp0.start()
        cp1 = pltpu.make_async_copy(big1_src.at[pl.ds(start, EB)], b1_ref, sem)
        cp1.start()
        cp2 = pltpu.make_async_copy(big2_src.at[pl.ds(start, EB)], b2_ref, sem)
        cp2.start()
        cp0.wait(); cp1.wait(); cp2.wait()
        dl = bd_ref[...] - base
        valid = jnp.logical_and(dl >= 0, dl < NB)
        S = jnp.logical_and(dl == lane_iota, valid)
        return start, S, S.astype(jnp.float32), valid

    q_blk = q_ref[...]

    def phase_a(j, m):
        start, S, Sf, valid = load_chunk(j, kg_ref, e_ref)
        qrows = jax.lax.dot_general(Sf, q_blk, (((1,), (0,)), ((), ())),
                                    preferred_element_type=jnp.float32)
        kj = b1_ref[...] + b2_ref[...]
        prod = qrows * kj
        cols = [jnp.sum(prod[:, h * C:(h + 1) * C], axis=1, keepdims=True)
                for h in range(H)]
        alpha = jnp.concatenate(cols, axis=1) * _INV_SQRT_C
        ba_ref[...] = alpha
        cpa = pltpu.make_async_copy(ba_ref, adump_ref.at[pl.ds(start, EB)], sem)
        cpa.start(); cpa.wait()
        new_m = []
        for h in range(H):
            mh = jnp.max(jnp.where(S, alpha[:, h:h + 1], _NEG),
                         axis=0, keepdims=True)
            new_m.append(jnp.maximum(m[h], mh))
        return tuple(new_m)

    m0 = tuple(jnp.full((1, NB), _NEG, jnp.float32) for _ in range(H))
    m = lax.fori_loop(0, nch, phase_a, m0)
    mrow4 = jnp.concatenate([jnp.where(mh < np.float32(-0.9e30), 0.0, mh)
                             for mh in m], axis=0)
    eye = (jax.lax.broadcasted_iota(jnp.int32, (NB, NB), 0)
           == jax.lax.broadcasted_iota(jnp.int32, (NB, NB), 1)).astype(jnp.float32)
    m_cols = jax.lax.dot_general(eye, mrow4, (((1,), (1,)), ((), ())),
                                 preferred_element_type=jnp.float32,
                                 precision=jax.lax.Precision.HIGHEST)

    def phase_b(j, carry):
        acc, den = carry
        start, S, Sf, valid = load_chunk(j, vg_ref, e_ref)
        cpa = pltpu.make_async_copy(adump_ref.at[pl.ds(start, EB)], ba_ref, sem)
        cpa.start(); cpa.wait()
        alpha = ba_ref[...]
        mrows = jax.lax.dot_general(Sf, m_cols, (((1,), (0,)), ((), ())),
                                    preferred_element_type=jnp.float32,
                                    precision=jax.lax.Precision.HIGHEST)
        ex = jnp.exp(jnp.minimum(alpha - mrows, 0.0))
        ex = jnp.where(valid, ex, 0.0)
        vj = b1_ref[...] + b2_ref[...]
        wv = jnp.concatenate(
            [vj[:, h * C:(h + 1) * C] * ex[:, h:h + 1] for h in range(H)], axis=1)
        wv = jnp.where(valid, wv, 0.0)
        den = den + jax.lax.dot_general(Sf, ex, (((0,), (0,)), ((), ())),
                                        preferred_element_type=jnp.float32,
                                        precision=jax.lax.Precision.HIGHEST)
        acc = acc + jax.lax.dot_general(Sf, wv, (((0,), (0,)), ((), ())),
                                        preferred_element_type=jnp.float32)
        return acc, den

    acc0 = jnp.zeros((NB, HC), jnp.float32)
    den0 = jnp.zeros((NB, H), jnp.float32)
    acc, den = lax.fori_loop(0, nch, phase_b, (acc0, den0))
    deninv = 1.0 / (den + np.float32(1e-16))
    out = jnp.concatenate(
        [acc[:, h * C:(h + 1) * C] * deninv[:, h:h + 1] for h in range(H)], axis=1)
    o_ref[...] = out + s_ref[...]


def _attention(offs, q, s_skip, dsts_pad, kg, vg, e_s):
    any_spec = pl.BlockSpec(memory_space=pl.ANY)
    return pl.pallas_call(
        _attn_kernel,
        grid=(NBLK,),
        in_specs=[
            pl.BlockSpec(memory_space=pltpu.SMEM),
            pl.BlockSpec((NB, HC), lambda i: (i, 0)),
            pl.BlockSpec((NB, HC), lambda i: (i, 0)),
            any_spec, any_spec, any_spec, any_spec,
        ],
        out_specs=[pl.BlockSpec((NB, HC), lambda i: (i, 0)), any_spec],
        out_shape=[jax.ShapeDtypeStruct((N, HC), jnp.float32),
                   jax.ShapeDtypeStruct((E_PAD, H), jnp.float32)],
        scratch_shapes=[
            pltpu.VMEM((EB, 1), jnp.int32),
            pltpu.VMEM((EB, HC), jnp.float32),
            pltpu.VMEM((EB, HC), jnp.float32),
            pltpu.VMEM((EB, H), jnp.float32),
            pltpu.SemaphoreType.DMA,
        ],
    )(offs, q, s_skip, dsts_pad, kg, vg, e_s)[0]


def _out_mm_kernel(h_ref, w_ref, b_ref, o_ref):
    o_ref[...] = jnp.dot(h_ref[...], w_ref[...],
                         preferred_element_type=jnp.float32) + b_ref[...]


def _out_matmul(h, w, b):
    blk = 1000
    return pl.pallas_call(
        _out_mm_kernel,
        grid=(N // blk,),
        in_specs=[
            pl.BlockSpec((blk, HC), lambda i: (i, 0)),
            pl.BlockSpec((HC, C), lambda i: (0, 0)),
            pl.BlockSpec((1, C), lambda i: (0, 0)),
        ],
        out_specs=pl.BlockSpec((blk, C), lambda i: (i, 0)),
        out_shape=jax.ShapeDtypeStruct((N, C), jnp.float32),
    )(h, w, b.reshape(1, C))


def kernel(event_type_ids, event_src_ids, event_dst_ids, event_embeddings,
           event_timestamps, graph_batch, graph_x, edge_index, edge_attr,
           edge_last_update, memory, te_w, te_b,
           Wq0, Wk0, Wv0, Ws0, We0, bq0, bk0, bv0, bs0, be0,
           Wq1, Wk1, Wv1, Ws1, We1, bq1, bk1, bv1, bs1, be1,
           Wq2, Wk2, Wv2, Ws2, We2, bq2, bk2, bv2, bs2, be2,
           Wout, bout):
    Wq = [Wq0, Wq1, Wq2]; Wk = [Wk0, Wk1, Wk2]; Wv = [Wv0, Wv1, Wv2]
    Ws = [Ws0, Ws1, Ws2]; We = [We0, We1, We2]
    bq = [bq0, bq1, bq2]; bk = [bk0, bk1, bk2]; bv = [bv0, bv1, bv2]
    bs = [bs0, bs1, bs2]; be = [be0, be1, be2]

    src = edge_index[0]
    dst = edge_index[1]

    perm = jnp.argsort(dst)
    dsts = dst[perm]
    srcs = src[perm]
    offs = jnp.searchsorted(dsts, jnp.arange(NBLK + 1, dtype=jnp.int32) * NB,
                            side='left').astype(jnp.int32)
    pad_i = jnp.zeros((E_PAD - E,), jnp.int32)
    perm_pad = jnp.concatenate([perm.astype(jnp.int32), pad_i])
    srcs_pad = jnp.concatenate([srcs.astype(jnp.int32), pad_i])
    dsts_pad = jnp.concatenate([dsts.astype(jnp.int32),
                                jnp.full((E_PAD - E,), N, jnp.int32)])

    x = jnp.concatenate([graph_x, memory], axis=-1)

    nts = _node_ts(graph_batch, event_timestamps)
    ea_ext = jnp.concatenate(
        [edge_attr, edge_last_update[:, None], jnp.zeros((E, 127), jnp.float32)],
        axis=1)
    ea_ext_s = _sc_gather(ea_ext, perm_pad, 256)
    nts_ext = jnp.concatenate([nts, jnp.zeros((N, 127), jnp.float32)], axis=1)
    ts_col = _sc_gather(nts_ext, srcs_pad, 128)[:, :1]

    h = None
    for i in range(NUM_BLOCK):
        xin = x if i == 0 else jnp.concatenate([h, x], axis=-1)
        q, k, v, s = _proj(xin, Wq[i], Wk[i], Wv[i], Ws[i],
                           bq[i], bk[i], bv[i], bs[i])
        kg = _sc_gather(k, srcs_pad, HC)
        vg = _sc_gather(v, srcs_pad, HC)
        e_s = _eproj(ea_ext_s, ts_col, te_w, te_b, We[i], be[i])
        h = _attention(offs, q, s, dsts_pad.reshape(E_PAD, 1), kg, vg, e_s)

    return _out_matmul(h, Wout, bout)


# speedup vs baseline: 1.2443x; 1.2443x over previous
import functools
import numpy as np
import jax
import jax.numpy as jnp
from jax import lax
from jax.experimental import pallas as pl
from jax.experimental.pallas import tpu as pltpu
from jax.experimental.pallas import tpu_sc as plsc

H = 4
C = 128
TE = 32
EE = 128
MEM = 128
N = 10000
E = 160000
B = 64
NODE_DIM = EE + MEM
EDGE_DIM = TE + EE
NUM_BLOCK = 3
HC = H * C

NW = 32
EB = 512
NB = 400
NBLK = N // NB
E_PAD = E + EB
PER_W = E_PAD // NW
GCH = 200
G_FULL = PER_W // GCH
G_TAIL = PER_W - G_FULL * GCH

def _sc_gather(table, idx, D):
    @functools.partial(
        pl.kernel,
        mesh=plsc.VectorSubcoreMesh(core_axis_name="c", subcore_axis_name="s"),
        out_type=jax.ShapeDtypeStruct((E_PAD, D), jnp.float32),
        scratch_types=[
            pltpu.VMEM((GCH,), jnp.int32),
            pltpu.VMEM((GCH, D), jnp.float32),
            pltpu.VMEM((G_TAIL,), jnp.int32),
            pltpu.VMEM((G_TAIL, D), jnp.float32),
            pltpu.SemaphoreType.DMA,
        ],
    )
    def k(table_hbm, idx_hbm, out_hbm, idx_v, rows_v, idx_t, rows_t, sem):
        wid = lax.axis_index("s") * 2 + lax.axis_index("c")
        base = wid * PER_W

        def body(j, carry):
            off = base + j * GCH
            pltpu.sync_copy(idx_hbm.at[pl.ds(off, GCH)], idx_v)
            pltpu.async_copy(table_hbm.at[idx_v], rows_v, sem).wait()
            pltpu.sync_copy(rows_v, out_hbm.at[pl.ds(off, GCH)])
            return carry

        lax.fori_loop(0, G_FULL, body, 0)
        toff = base + G_FULL * GCH
        pltpu.sync_copy(idx_hbm.at[pl.ds(toff, G_TAIL)], idx_t)
        pltpu.async_copy(table_hbm.at[idx_t], rows_t, sem).wait()
        pltpu.sync_copy(rows_t, out_hbm.at[pl.ds(toff, G_TAIL)])

    return k(table, idx)


def _node_ts_kernel(b_ref, t_ref, o_ref):
    bid = jax.lax.broadcasted_iota(jnp.int32, (1000, B), 1)
    sel = (b_ref[...] == bid).astype(jnp.float32)
    o_ref[...] = jnp.sum(sel * t_ref[...], axis=1, keepdims=True)


def _node_ts(batch, ts):
    return pl.pallas_call(
        _node_ts_kernel,
        grid=(N // 1000,),
        in_specs=[
            pl.BlockSpec((1000, 1), lambda i: (i, 0)),
            pl.BlockSpec((1, B), lambda i: (0, 0)),
        ],
        out_specs=pl.BlockSpec((1000, 1), lambda i: (i, 0)),
        out_shape=jax.ShapeDtypeStruct((N, 1), jnp.float32),
    )(batch.reshape(N, 1), ts.reshape(1, B))


def _proj_kernel(x_ref, wq_ref, wk_ref, wv_ref, ws_ref, bq_ref, bk_ref,
                 bv_ref, bs_ref, q_ref, k_ref, v_ref, s_ref):
    x = x_ref[...]
    f32 = jnp.float32
    q_ref[...] = jnp.dot(x, wq_ref[...], preferred_element_type=f32) + bq_ref[...]
    k_ref[...] = jnp.dot(x, wk_ref[...], preferred_element_type=f32) + bk_ref[...]
    v_ref[...] = jnp.dot(x, wv_ref[...], preferred_element_type=f32) + bv_ref[...]
    s_ref[...] = jnp.dot(x, ws_ref[...], preferred_element_type=f32) + bs_ref[...]


def _proj(x, wq, wk, wv, ws, bq, bk, bv, bs):
    ind = x.shape[1]
    blk = 1000
    wspec = pl.BlockSpec((ind, HC), lambda i: (0, 0))
    bspec = pl.BlockSpec((1, HC), lambda i: (0, 0))
    ospec = pl.BlockSpec((blk, HC), lambda i: (i, 0))
    oshape = jax.ShapeDtypeStruct((N, HC), jnp.float32)
    return pl.pallas_call(
        _proj_kernel,
        grid=(N // blk,),
        in_specs=[pl.BlockSpec((blk, ind), lambda i: (i, 0)),
                  wspec, wspec, wspec, wspec, bspec, bspec, bspec, bspec],
        out_specs=[ospec, ospec, ospec, ospec],
        out_shape=[oshape, oshape, oshape, oshape],
    )(x, wq, wk, wv, ws, bq.reshape(1, HC), bk.reshape(1, HC),
      bv.reshape(1, HC), bs.reshape(1, HC))


EB2 = 1056


def _eproj_body(ea_ref, ts_ref, tw_ref, tb_ref, wt_ref, wa_ref, be_ref, o_ref):
    ea = ea_ref[...]
    rel_t = ts_ref[...] - ea[:, EE:EE + 1]
    rel_emb = jnp.cos(rel_t * tw_ref[...] + tb_ref[...])
    o_ref[...] = (jnp.dot(rel_emb, wt_ref[...], preferred_element_type=jnp.float32)
                  + jnp.dot(ea[:, :EE], wa_ref[...], preferred_element_type=jnp.float32)
                  + be_ref[...])


def _eproj(ea_ext_s, ts_col, te_w, te_b, We, be):
    return pl.pallas_call(
        _eproj_body,
        grid=(E_PAD // EB2,),
        in_specs=[
            pl.BlockSpec((EB2, 256), lambda i: (i, 0)),
            pl.BlockSpec((EB2, 1), lambda i: (i, 0)),
            pl.BlockSpec((1, TE), lambda i: (0, 0)),
            pl.BlockSpec((1, TE), lambda i: (0, 0)),
            pl.BlockSpec((TE, HC), lambda i: (0, 0)),
            pl.BlockSpec((EE, HC), lambda i: (0, 0)),
            pl.BlockSpec((1, HC), lambda i: (0, 0)),
        ],
        out_specs=pl.BlockSpec((EB2, HC), lambda i: (i, 0)),
        out_shape=jax.ShapeDtypeStruct((E_PAD, HC), jnp.float32),
    )(ea_ext_s, ts_col, te_w.reshape(1, TE), te_b.reshape(1, TE),
      We[:TE], We[TE:], be.reshape(1, HC))


_INV_SQRT_C = np.float32(1.0 / np.sqrt(C))
_NEG = np.float32(-1e30)


def _attn_kernel(offs_ref, q_ref, s_ref, dst_ref, kg_ref, vg_ref, e_ref,
                 o_ref, adump_ref,
                 bd_ref, b1_ref, b2_ref, ba_ref, sem):
    nb = pl.program_id(0)
    off0 = offs_ref[nb]
    off1 = offs_ref[nb + 1]
    astart = (off0 // 8) * 8
    nch = (off1 - astart + EB - 1) // EB
    base = nb * NB

    lane_iota = jax.lax.broadcasted_iota(jnp.int32, (EB, NB), 1)

    def load_chunk(j, big1_src, big2_src):
        start = astart + j * EB
        cp0 = pltpu.make_async_copy(dst_ref.at[pl.ds(start, EB)], bd_ref, sem)
        cp0.start()
        cp1 = pltpu.make_async_copy(big1_src.at[pl.ds(start, EB)], b1_ref, sem)
        cp1.start()
        cp2 = pltpu.make_async_copy(big2_src.at[pl.ds(start, EB)], b2_ref, sem)
        cp2.start()
        cp0.wait(); cp1.wait(); cp2.wait()
        dl = bd_ref[...] - base
        valid = jnp.logical_and(dl >= 0, dl < NB)
        S = jnp.logical_and(dl == lane_iota, valid)
        return start, S, S.astype(jnp.float32), valid

    q_blk = q_ref[...]

    def phase_a(j, m):
        start, S, Sf, valid = load_chunk(j, kg_ref, e_ref)
        qrows = jax.lax.dot_general(Sf, q_blk, (((1,), (0,)), ((), ())),
                                    preferred_element_type=jnp.float32)
        kj = b1_ref[...] + b2_ref[...]
        prod = qrows * kj
        cols = [jnp.sum(prod[:, h * C:(h + 1) * C], axis=1, keepdims=True)
                for h in range(H)]
        alpha = jnp.concatenate(cols, axis=1) * _INV_SQRT_C
        ba_ref[...] = alpha
        cpa = pltpu.make_async_copy(ba_ref, adump_ref.at[pl.ds(start, EB)], sem)
        cpa.start(); cpa.wait()
        new_m = []
        for h in range(H):
            mh = jnp.max(jnp.where(S, alpha[:, h:h + 1], _NEG),
                         axis=0, keepdims=True)
            new_m.append(jnp.maximum(m[h], mh))
        return tuple(new_m)

    m0 = tuple(jnp.full((1, NB), _NEG, jnp.float32) for _ in range(H))
    m = lax.fori_loop(0, nch, phase_a, m0)
    mrow4 = jnp.concatenate([jnp.where(mh < np.float32(-0.9e30), 0.0, mh)
                             for mh in m], axis=0)
    eye = (jax.lax.broadcasted_iota(jnp.int32, (NB, NB), 0)
           == jax.lax.broadcasted_iota(jnp.int32, (NB, NB), 1)).astype(jnp.float32)
    m_cols = jax.lax.dot_general(eye, mrow4, (((1,), (1,)), ((), ())),
                                 preferred_element_type=jnp.float32)

    def phase_b(j, carry):
        acc, den = carry
        start, S, Sf, valid = load_chunk(j, vg_ref, e_ref)
        cpa = pltpu.make_async_copy(adump_ref.at[pl.ds(start, EB)], ba_ref, sem)
        cpa.start(); cpa.wait()
        alpha = ba_ref[...]
        mrows = jax.lax.dot_general(Sf, m_cols, (((1,), (0,)), ((), ())),
                                    preferred_element_type=jnp.float32)
        ex = jnp.exp(jnp.minimum(alpha - mrows, 0.0))
        ex = jnp.where(valid, ex, 0.0)
        vj = b1_ref[...] + b2_ref[...]
        wv = jnp.concatenate(
            [vj[:, h * C:(h + 1) * C] * ex[:, h:h + 1] for h in range(H)], axis=1)
        wv = jnp.where(valid, wv, 0.0)
        den = den + jax.lax.dot_general(Sf, ex, (((0,), (0,)), ((), ())),
                                        preferred_element_type=jnp.float32)
        acc = acc + jax.lax.dot_general(Sf, wv, (((0,), (0,)), ((), ())),
                                        preferred_element_type=jnp.float32)
        return acc, den

    acc0 = jnp.zeros((NB, HC), jnp.float32)
    den0 = jnp.zeros((NB, H), jnp.float32)
    acc, den = lax.fori_loop(0, nch, phase_b, (acc0, den0))
    deninv = 1.0 / (den + np.float32(1e-16))
    out = jnp.concatenate(
        [acc[:, h * C:(h + 1) * C] * deninv[:, h:h + 1] for h in range(H)], axis=1)
    o_ref[...] = out + s_ref[...]


def _attention(offs, q, s_skip, dsts_pad, kg, vg, e_s):
    any_spec = pl.BlockSpec(memory_space=pl.ANY)
    return pl.pallas_call(
        _attn_kernel,
        grid=(NBLK,),
        in_specs=[
            pl.BlockSpec(memory_space=pltpu.SMEM),
            pl.BlockSpec((NB, HC), lambda i: (i, 0)),
            pl.BlockSpec((NB, HC), lambda i: (i, 0)),
            any_spec, any_spec, any_spec, any_spec,
        ],
        out_specs=[pl.BlockSpec((NB, HC), lambda i: (i, 0)), any_spec],
        out_shape=[jax.ShapeDtypeStruct((N, HC), jnp.float32),
                   jax.ShapeDtypeStruct((E_PAD, H), jnp.float32)],
        scratch_shapes=[
            pltpu.VMEM((EB, 1), jnp.int32),
            pltpu.VMEM((EB, HC), jnp.float32),
            pltpu.VMEM((EB, HC), jnp.float32),
            pltpu.VMEM((EB, H), jnp.float32),
            pltpu.SemaphoreType.DMA,
        ],
    )(offs, q, s_skip, dsts_pad, kg, vg, e_s)[0]


def _out_mm_kernel(h_ref, w_ref, b_ref, o_ref):
    o_ref[...] = jnp.dot(h_ref[...], w_ref[...],
                         preferred_element_type=jnp.float32) + b_ref[...]


def _out_matmul(h, w, b):
    blk = 1000
    return pl.pallas_call(
        _out_mm_kernel,
        grid=(N // blk,),
        in_specs=[
            pl.BlockSpec((blk, HC), lambda i: (i, 0)),
            pl.BlockSpec((HC, C), lambda i: (0, 0)),
            pl.BlockSpec((1, C), lambda i: (0, 0)),
        ],
        out_specs=pl.BlockSpec((blk, C), lambda i: (i, 0)),
        out_shape=jax.ShapeDtypeStruct((N, C), jnp.float32),
    )(h, w, b.reshape(1, C))


def kernel(event_type_ids, event_src_ids, event_dst_ids, event_embeddings,
           event_timestamps, graph_batch, graph_x, edge_index, edge_attr,
           edge_last_update, memory, te_w, te_b,
           Wq0, Wk0, Wv0, Ws0, We0, bq0, bk0, bv0, bs0, be0,
           Wq1, Wk1, Wv1, Ws1, We1, bq1, bk1, bv1, bs1, be1,
           Wq2, Wk2, Wv2, Ws2, We2, bq2, bk2, bv2, bs2, be2,
           Wout, bout):
    Wq = [Wq0, Wq1, Wq2]; Wk = [Wk0, Wk1, Wk2]; Wv = [Wv0, Wv1, Wv2]
    Ws = [Ws0, Ws1, Ws2]; We = [We0, We1, We2]
    bq = [bq0, bq1, bq2]; bk = [bk0, bk1, bk2]; bv = [bv0, bv1, bv2]
    bs = [bs0, bs1, bs2]; be = [be0, be1, be2]

    src = edge_index[0]
    dst = edge_index[1]

    perm = jnp.argsort(dst)
    dsts = dst[perm]
    srcs = src[perm]
    offs = jnp.searchsorted(dsts, jnp.arange(NBLK + 1, dtype=jnp.int32) * NB,
                            side='left').astype(jnp.int32)
    pad_i = jnp.zeros((E_PAD - E,), jnp.int32)
    perm_pad = jnp.concatenate([perm.astype(jnp.int32), pad_i])
    srcs_pad = jnp.concatenate([srcs.astype(jnp.int32), pad_i])
    dsts_pad = jnp.concatenate([dsts.astype(jnp.int32),
                                jnp.full((E_PAD - E,), N, jnp.int32)])

    x = jnp.concatenate([graph_x, memory], axis=-1)

    nts = _node_ts(graph_batch, event_timestamps)
    ea_ext = jnp.concatenate(
        [edge_attr, edge_last_update[:, None], jnp.zeros((E, 127), jnp.float32)],
        axis=1)
    ea_ext_s = _sc_gather(ea_ext, perm_pad, 256)
    nts_ext = jnp.concatenate([nts, jnp.zeros((N, 127), jnp.float32)], axis=1)
    ts_col = _sc_gather(nts_ext, srcs_pad, 128)[:, :1]

    h = None
    for i in range(NUM_BLOCK):
        xin = x if i == 0 else jnp.concatenate([h, x], axis=-1)
        q, k, v, s = _proj(xin, Wq[i], Wk[i], Wv[i], Ws[i],
                           bq[i], bk[i], bv[i], bs[i])
        kg = _sc_gather(k, srcs_pad, HC)
        vg = _sc_gather(v, srcs_pad, HC)
        e_s = _eproj(ea_ext_s, ts_col, te_w, te_b, We[i], be[i])
        h = _attention(offs, q, s, dsts_pad.reshape(E_PAD, 1), kg, vg, e_s)

    return _out_matmul(h, Wout, bout)
